# baseline (speedup 1.0000x reference)
import functools

import jax
import jax.numpy as jnp
from jax import lax
from jax.experimental import pallas as pl
from jax.experimental.pallas import tpu as pltpu
from jax.experimental.pallas import tpu_sc as plsc

_D = 2048
_MAX_LEN = 8192
_VOCAB = 2 * _MAX_LEN

_ROWS_PER_BLK = 512
_MLP_BLKS = _MAX_LEN // _ROWS_PER_BLK


def _table_body(
    emb_hbm, w1r, b1r, w2t, b2r, w3t, b3r, out_hbm,
    buf0, buf1, csem, osem0, osem1,
):
    i = pl.program_id(0)

    @pl.when(i == 0)
    def _():
        pltpu.make_async_copy(
            emb_hbm, out_hbm.at[pl.ds(0, _MAX_LEN)], csem
        ).start()

    def mlp(buf):
        base = _MAX_LEN + i * _ROWS_PER_BLK
        x = (base + lax.broadcasted_iota(jnp.int32, (_ROWS_PER_BLK, 1), 0)).astype(
            jnp.float32
        )
        h1 = jnp.maximum(x * w1r[...] + b1r[...], 0.0)
        h2 = jnp.maximum(
            jnp.dot(h1, w2t[...], preferred_element_type=jnp.float32) + b2r[...],
            0.0,
        )
        buf[...] = (
            jnp.dot(h2, w3t[...], preferred_element_type=jnp.float32) + b3r[...]
        )

    def out_rows():
        return out_hbm.at[pl.ds(_MAX_LEN + i * _ROWS_PER_BLK, _ROWS_PER_BLK)]

    @pl.when(i % 2 == 0)
    def _():
        @pl.when(i >= 2)
        def _():
            pltpu.make_async_copy(buf0, out_rows(), osem0).wait()

        mlp(buf0)
        pltpu.make_async_copy(buf0, out_rows(), osem0).start()

    @pl.when(i % 2 == 1)
    def _():
        @pl.when(i >= 2)
        def _():
            pltpu.make_async_copy(buf1, out_rows(), osem1).wait()

        mlp(buf1)
        pltpu.make_async_copy(buf1, out_rows(), osem1).start()

    @pl.when(i == _MLP_BLKS - 1)
    def _():
        pltpu.make_async_copy(buf0, out_rows(), osem0).wait()
        pltpu.make_async_copy(buf1, out_rows(), osem1).wait()
        pltpu.make_async_copy(
            emb_hbm, out_hbm.at[pl.ds(0, _MAX_LEN)], csem
        ).wait()


def _build_combined_table(pos_embeddings, w1r, b1r, w2t, b2r, w3t, b3r):
    return pl.pallas_call(
        _table_body,
        grid=(_MLP_BLKS,),
        in_specs=[
            pl.BlockSpec(memory_space=pltpu.MemorySpace.HBM),
            pl.BlockSpec((1, 64), lambda i: (0, 0)),
            pl.BlockSpec((1, 64), lambda i: (0, 0)),
            pl.BlockSpec((64, 128), lambda i: (0, 0)),
            pl.BlockSpec((1, 128), lambda i: (0, 0)),
            pl.BlockSpec((128, _D), lambda i: (0, 0)),
            pl.BlockSpec((1, _D), lambda i: (0, 0)),
        ],
        out_specs=pl.BlockSpec(memory_space=pltpu.MemorySpace.HBM),
        out_shape=jax.ShapeDtypeStruct((_VOCAB, _D), jnp.float32),
        scratch_shapes=[
            pltpu.VMEM((_ROWS_PER_BLK, _D), jnp.float32),
            pltpu.VMEM((_ROWS_PER_BLK, _D), jnp.float32),
            pltpu.SemaphoreType.DMA,
            pltpu.SemaphoreType.DMA,
            pltpu.SemaphoreType.DMA,
        ],
    )(pos_embeddings, w1r, b1r, w2t, b2r, w3t, b3r)


_NC = 2
_NS = 16
_NW = _NC * _NS

_N_TOK = 4 * 8192
_TOK_PER_W = _N_TOK // _NW
_CH = 8
_NCH = _TOK_PER_W // _CH
_DEPTH = 4


def _gather_body(table_hbm, idx_hbm, out_hbm, idx_v, rows, gsems, ssems):
    wid = lax.axis_index("s") * _NC + lax.axis_index("c")
    base = wid * _TOK_PER_W
    pltpu.sync_copy(idx_hbm.at[pl.ds(base, _TOK_PER_W)], idx_v)

    def start_gather(c, slot):
        pltpu.async_copy(
            table_hbm.at[idx_v.at[pl.ds(c * _CH, _CH)]], rows[slot], gsems[slot]
        )

    def drain_gather(slot):
        pltpu.make_async_copy(
            table_hbm.at[pl.ds(0, _CH)], rows[slot], gsems[slot]
        ).wait()

    def start_scatter(c, slot):
        pltpu.async_copy(
            rows[slot], out_hbm.at[pl.ds(base + c * _CH, _CH)], ssems[slot]
        )

    def drain_scatter(slot):
        pltpu.make_async_copy(
            table_hbm.at[pl.ds(0, _CH)], rows[slot], ssems[slot]
        ).wait()

    start_gather(0, 0)
    start_gather(1, 1)

    def step(g, carry):
        for j in range(_DEPTH):
            c = g * _DEPTH + j
            s = j
            t = (j + 2) % _DEPTH

            @pl.when(c >= 2)
            def _():
                drain_scatter(t)

            @pl.when(c + 2 < _NCH)
            def _():
                start_gather(c + 2, t)

            drain_gather(s)
            start_scatter(c, s)

        return carry

    lax.fori_loop(0, _NCH // _DEPTH, step, 0)

    drain_scatter((_NCH - 2) % _DEPTH)
    drain_scatter((_NCH - 1) % _DEPTH)


def _sc_gather(table, idx):
    mesh = plsc.VectorSubcoreMesh(core_axis_name="c", subcore_axis_name="s")
    f = functools.partial(
        pl.kernel,
        out_type=jax.ShapeDtypeStruct((_N_TOK, _D), jnp.float32),
        mesh=mesh,
        scratch_types=[
            pltpu.VMEM((_TOK_PER_W,), jnp.int32),
            [pltpu.VMEM((_CH, _D), jnp.float32) for _ in range(_DEPTH)],
            [pltpu.SemaphoreType.DMA for _ in range(_DEPTH)],
            [pltpu.SemaphoreType.DMA for _ in range(_DEPTH)],
        ],
    )(_gather_body)
    return f(table, idx)


def kernel(pos, pos_embeddings, W1, b1, W2, b2, W3, b3):
    batch, seq = pos.shape
    idx = pos.reshape(-1).astype(jnp.int32)
    w1r = W1.reshape(1, 64)
    b1r = b1.reshape(1, 64)
    w2t = W2.T
    b2r = b2.reshape(1, 128)
    w3t = W3.T
    b3r = b3.reshape(1, _D)
    table = _build_combined_table(pos_embeddings, w1r, b1r, w2t, b2r, w3t, b3r)
    out = _sc_gather(table, idx)
    return out.reshape(batch, seq, _D)


# speedup vs baseline: 7.9899x; 7.9899x over previous
import functools

import jax
import jax.numpy as jnp
from jax import lax
from jax.experimental import pallas as pl
from jax.experimental.pallas import tpu as pltpu
from jax.experimental.pallas import tpu_sc as plsc

_D = 2048
_MAX_LEN = 8192
_VOCAB = 2 * _MAX_LEN

_ROWS_PER_BLK = 1024
_NUM_BLKS = _VOCAB // _ROWS_PER_BLK
_COPY_BLKS = _MAX_LEN // _ROWS_PER_BLK


def _table_body(emb_hbm, w1r, b1r, w2t, b2r, w3t, b3r, out_ref, sem):
    i = pl.program_id(0)

    @pl.when(i < _COPY_BLKS)
    def _copy():
        cp = pltpu.make_async_copy(
            emb_hbm.at[pl.ds(i * _ROWS_PER_BLK, _ROWS_PER_BLK)], out_ref, sem
        )
        cp.start()
        cp.wait()

    @pl.when(i >= _COPY_BLKS)
    def _mlp():
        base = i * _ROWS_PER_BLK
        x = (base + lax.broadcasted_iota(jnp.int32, (_ROWS_PER_BLK, 1), 0)).astype(
            jnp.float32
        )
        h1 = jnp.maximum(x * w1r[...] + b1r[...], 0.0)
        h2 = jnp.maximum(
            jnp.dot(h1, w2t[...], preferred_element_type=jnp.float32) + b2r[...],
            0.0,
        )
        out_ref[...] = (
            jnp.dot(h2, w3t[...], preferred_element_type=jnp.float32) + b3r[...]
        )


def _build_combined_table(pos_embeddings, w1r, b1r, w2t, b2r, w3t, b3r):
    return pl.pallas_call(
        _table_body,
        grid=(_NUM_BLKS,),
        in_specs=[
            pl.BlockSpec(memory_space=pltpu.MemorySpace.HBM),
            pl.BlockSpec((1, 64), lambda i: (0, 0)),
            pl.BlockSpec((1, 64), lambda i: (0, 0)),
            pl.BlockSpec((64, 128), lambda i: (0, 0)),
            pl.BlockSpec((1, 128), lambda i: (0, 0)),
            pl.BlockSpec((128, _D), lambda i: (0, 0)),
            pl.BlockSpec((1, _D), lambda i: (0, 0)),
        ],
        out_specs=pl.BlockSpec((_ROWS_PER_BLK, _D), lambda i: (i, 0)),
        out_shape=jax.ShapeDtypeStruct((_VOCAB, _D), jnp.float32),
        scratch_shapes=[pltpu.SemaphoreType.DMA],
    )(pos_embeddings, w1r, b1r, w2t, b2r, w3t, b3r)


_NC = 2
_NS = 16
_NW = _NC * _NS

_N_TOK = 4 * 8192
_TOK_PER_W = _N_TOK // _NW
_CH = 8
_NCH = _TOK_PER_W // _CH
_DEPTH = 4


def _gather_body(table_hbm, idx_hbm, out_hbm, idx_v, rows, gsems, ssems):
    wid = lax.axis_index("s") * _NC + lax.axis_index("c")
    base = wid * _TOK_PER_W
    pltpu.sync_copy(idx_hbm.at[pl.ds(base, _TOK_PER_W)], idx_v)

    def start_gather(c, slot):
        pltpu.async_copy(
            table_hbm.at[idx_v.at[pl.ds(c * _CH, _CH)]], rows[slot], gsems[slot]
        )

    def drain_gather(slot):
        pltpu.make_async_copy(
            table_hbm.at[pl.ds(0, _CH)], rows[slot], gsems[slot]
        ).wait()

    def start_scatter(c, slot):
        pltpu.async_copy(
            rows[slot], out_hbm.at[pl.ds(base + c * _CH, _CH)], ssems[slot]
        )

    def drain_scatter(slot):
        pltpu.make_async_copy(
            table_hbm.at[pl.ds(0, _CH)], rows[slot], ssems[slot]
        ).wait()

    start_gather(0, 0)
    start_gather(1, 1)

    def step(g, carry):
        for j in range(_DEPTH):
            c = g * _DEPTH + j
            s = j
            t = (j + 2) % _DEPTH

            @pl.when(c >= 2)
            def _():
                drain_scatter(t)

            @pl.when(c + 2 < _NCH)
            def _():
                start_gather(c + 2, t)

            drain_gather(s)
            start_scatter(c, s)

        return carry

    lax.fori_loop(0, _NCH // _DEPTH, step, 0)

    drain_scatter((_NCH - 2) % _DEPTH)
    drain_scatter((_NCH - 1) % _DEPTH)


def _sc_gather(table, idx):
    mesh = plsc.VectorSubcoreMesh(core_axis_name="c", subcore_axis_name="s")
    f = functools.partial(
        pl.kernel,
        out_type=jax.ShapeDtypeStruct((_N_TOK, _D), jnp.float32),
        mesh=mesh,
        scratch_types=[
            pltpu.VMEM((_TOK_PER_W,), jnp.int32),
            [pltpu.VMEM((_CH, _D), jnp.float32) for _ in range(_DEPTH)],
            [pltpu.SemaphoreType.DMA for _ in range(_DEPTH)],
            [pltpu.SemaphoreType.DMA for _ in range(_DEPTH)],
        ],
    )(_gather_body)
    return f(table, idx)


def kernel(pos, pos_embeddings, W1, b1, W2, b2, W3, b3):
    batch, seq = pos.shape
    idx = pos.reshape(-1).astype(jnp.int32)
    w1r = W1.reshape(1, 64)
    b1r = b1.reshape(1, 64)
    w2t = W2.T
    b2r = b2.reshape(1, 128)
    w3t = W3.T
    b3r = b3.reshape(1, _D)
    table = _build_combined_table(pos_embeddings, w1r, b1r, w2t, b2r, w3t, b3r)
    out = _sc_gather(table, idx)
    return out.reshape(batch, seq, _D)


# speedup vs baseline: 8.1079x; 1.0148x over previous
import functools

import jax
import jax.numpy as jnp
from jax import lax
from jax.experimental import pallas as pl
from jax.experimental.pallas import tpu as pltpu
from jax.experimental.pallas import tpu_sc as plsc

_D = 2048
_MAX_LEN = 8192
_VOCAB = 2 * _MAX_LEN

_ROWS_PER_BLK = 2048
_NUM_BLKS = _VOCAB // _ROWS_PER_BLK
_COPY_BLKS = _MAX_LEN // _ROWS_PER_BLK


def _table_body(emb_hbm, w1r, b1r, w2t, b2r, w3t, b3r, out_ref, sem):
    i = pl.program_id(0)

    @pl.when(i < _COPY_BLKS)
    def _copy():
        cp = pltpu.make_async_copy(
            emb_hbm.at[pl.ds(i * _ROWS_PER_BLK, _ROWS_PER_BLK)], out_ref, sem
        )
        cp.start()
        cp.wait()

    @pl.when(i >= _COPY_BLKS)
    def _mlp():
        base = i * _ROWS_PER_BLK
        x = (base + lax.broadcasted_iota(jnp.int32, (_ROWS_PER_BLK, 1), 0)).astype(
            jnp.float32
        )
        h1 = jnp.maximum(x * w1r[...] + b1r[...], 0.0)
        h2 = jnp.maximum(
            jnp.dot(h1, w2t[...], preferred_element_type=jnp.float32) + b2r[...],
            0.0,
        )
        out_ref[...] = (
            jnp.dot(h2, w3t[...], preferred_element_type=jnp.float32) + b3r[...]
        )


def _build_combined_table(pos_embeddings, w1r, b1r, w2t, b2r, w3t, b3r):
    return pl.pallas_call(
        _table_body,
        grid=(_NUM_BLKS,),
        in_specs=[
            pl.BlockSpec(memory_space=pltpu.MemorySpace.HBM),
            pl.BlockSpec((1, 64), lambda i: (0, 0)),
            pl.BlockSpec((1, 64), lambda i: (0, 0)),
            pl.BlockSpec((64, 128), lambda i: (0, 0)),
            pl.BlockSpec((1, 128), lambda i: (0, 0)),
            pl.BlockSpec((128, _D), lambda i: (0, 0)),
            pl.BlockSpec((1, _D), lambda i: (0, 0)),
        ],
        out_specs=pl.BlockSpec((_ROWS_PER_BLK, _D), lambda i: (i, 0)),
        out_shape=jax.ShapeDtypeStruct((_VOCAB, _D), jnp.float32),
        scratch_shapes=[pltpu.SemaphoreType.DMA],
    )(pos_embeddings, w1r, b1r, w2t, b2r, w3t, b3r)


_NC = 2
_NS = 16
_NW = _NC * _NS

_N_TOK = 4 * 8192
_TOK_PER_W = _N_TOK // _NW
_CH = 8
_NCH = _TOK_PER_W // _CH
_DEPTH = 4


def _gather_body(table_hbm, idx_hbm, out_hbm, idx_v, rows, gsems, ssems):
    wid = lax.axis_index("s") * _NC + lax.axis_index("c")
    base = wid * _TOK_PER_W
    pltpu.sync_copy(idx_hbm.at[pl.ds(base, _TOK_PER_W)], idx_v)

    def start_gather(c, slot):
        pltpu.async_copy(
            table_hbm.at[idx_v.at[pl.ds(c * _CH, _CH)]], rows[slot], gsems[slot]
        )

    def drain_gather(slot):
        pltpu.make_async_copy(
            table_hbm.at[pl.ds(0, _CH)], rows[slot], gsems[slot]
        ).wait()

    def start_scatter(c, slot):
        pltpu.async_copy(
            rows[slot], out_hbm.at[pl.ds(base + c * _CH, _CH)], ssems[slot]
        )

    def drain_scatter(slot):
        pltpu.make_async_copy(
            table_hbm.at[pl.ds(0, _CH)], rows[slot], ssems[slot]
        ).wait()

    start_gather(0, 0)
    start_gather(1, 1)

    def step(g, carry):
        for j in range(_DEPTH):
            c = g * _DEPTH + j
            s = j
            t = (j + 2) % _DEPTH

            @pl.when(c >= 2)
            def _():
                drain_scatter(t)

            @pl.when(c + 2 < _NCH)
            def _():
                start_gather(c + 2, t)

            drain_gather(s)
            start_scatter(c, s)

        return carry

    lax.fori_loop(0, _NCH // _DEPTH, step, 0)

    drain_scatter((_NCH - 2) % _DEPTH)
    drain_scatter((_NCH - 1) % _DEPTH)


def _sc_gather(table, idx):
    mesh = plsc.VectorSubcoreMesh(core_axis_name="c", subcore_axis_name="s")
    f = functools.partial(
        pl.kernel,
        out_type=jax.ShapeDtypeStruct((_N_TOK, _D), jnp.float32),
        mesh=mesh,
        scratch_types=[
            pltpu.VMEM((_TOK_PER_W,), jnp.int32),
            [pltpu.VMEM((_CH, _D), jnp.float32) for _ in range(_DEPTH)],
            [pltpu.SemaphoreType.DMA for _ in range(_DEPTH)],
            [pltpu.SemaphoreType.DMA for _ in range(_DEPTH)],
        ],
    )(_gather_body)
    return f(table, idx)


def kernel(pos, pos_embeddings, W1, b1, W2, b2, W3, b3):
    batch, seq = pos.shape
    idx = pos.reshape(-1).astype(jnp.int32)
    w1r = W1.reshape(1, 64)
    b1r = b1.reshape(1, 64)
    w2t = W2.T
    b2r = b2.reshape(1, 128)
    w3t = W3.T
    b3r = b3.reshape(1, _D)
    table = _build_combined_table(pos_embeddings, w1r, b1r, w2t, b2r, w3t, b3r)
    out = _sc_gather(table, idx)
    return out.reshape(batch, seq, _D)
